# Initial kernel scaffold; baseline (speedup 1.0000x reference)
import functools
import jax
import jax.numpy as jnp
from jax import lax
from jax.experimental import pallas as pl
from jax.experimental.pallas import tpu as pltpu
from jax.experimental.pallas import tpu_sc as plsc

NC, NS, L = 2, 16, 16
NW = NC * NS
B = 16384
BPW = B // NW
UD = 32
PD = 16
N_OCC = 22

_mesh = plsc.VectorSubcoreMesh(core_axis_name="c", subcore_axis_name="s")


@functools.partial(
    pl.kernel,
    out_type=jax.ShapeDtypeStruct((B, UD + PD), jnp.float32),
    mesh=_mesh,
    scratch_types=[
        pltpu.VMEM((BPW,), jnp.int32),
        pltpu.VMEM((BPW,), jnp.int32),
        pltpu.VMEM((BPW,), jnp.int32),
        pltpu.VMEM((BPW, UD), jnp.float32),
        pltpu.VMEM((BPW, PD), jnp.float32),
        pltpu.SemaphoreType.DMA,
        pltpu.SemaphoreType.DMA,
    ],
)
def _user_model_sc(uid_hbm, gid_hbm, oid_hbm, utab_hbm, ptab_hbm, out_hbm,
                   uidx_v, pidx_v, oidx_v, urows_v, prows_v, sem_u, sem_p):
    wid = lax.axis_index("s") * NC + lax.axis_index("c")
    base = wid * BPW

    pltpu.sync_copy(uid_hbm.at[pl.ds(base, BPW)], uidx_v)
    cp_u = pltpu.async_copy(utab_hbm.at[uidx_v], urows_v, sem_u)

    pltpu.sync_copy(gid_hbm.at[pl.ds(base, BPW)], pidx_v)
    pltpu.sync_copy(oid_hbm.at[pl.ds(base, BPW)], oidx_v)

    def pair_body(i, _):
        g = pidx_v[pl.ds(i * L, L)]
        o = oidx_v[pl.ds(i * L, L)]
        pidx_v[pl.ds(i * L, L)] = g * N_OCC + o
        return ()

    lax.fori_loop(0, BPW // L, pair_body, (), unroll=4)

    cp_p = pltpu.async_copy(ptab_hbm.at[pidx_v], prows_v, sem_p)

    cp_u.wait()
    pltpu.sync_copy(urows_v, out_hbm.at[pl.ds(base, BPW), pl.ds(0, UD)])
    cp_p.wait()
    pltpu.sync_copy(prows_v, out_hbm.at[pl.ds(base, BPW), pl.ds(UD, PD)])


def kernel(user_id, gender, occupation, user_table, gender_table,
           occupation_table):
    n_g = gender_table.shape[0]
    n_o = occupation_table.shape[0]
    pairs = jnp.concatenate([
        jnp.broadcast_to(gender_table[:, None, :], (n_g, n_o, 8)),
        jnp.broadcast_to(occupation_table[None, :, :], (n_g, n_o, 8)),
    ], axis=-1).reshape(n_g * n_o, PD)
    return _user_model_sc(user_id, gender, occupation, user_table, pairs)


# speedup vs baseline: 2.0241x; 2.0241x over previous
import functools
import jax
import jax.numpy as jnp
from jax import lax
from jax.experimental import pallas as pl
from jax.experimental.pallas import tpu as pltpu
from jax.experimental.pallas import tpu_sc as plsc

NC, NS, L = 2, 16, 16
NW = NC * NS
B = 16384
BPW = B // NW
UD = 32
PD = 16
N_OCC = 22

_mesh = plsc.VectorSubcoreMesh(core_axis_name="c", subcore_axis_name="s")


@functools.partial(
    pl.kernel,
    out_type=jax.ShapeDtypeStruct((B, UD + PD), jnp.float32),
    mesh=_mesh,
    compiler_params=pltpu.CompilerParams(use_tc_tiling_on_sc=False),
    scratch_types=[
        pltpu.VMEM((BPW,), jnp.int32),
        pltpu.VMEM((BPW,), jnp.int32),
        pltpu.VMEM((BPW,), jnp.int32),
        pltpu.VMEM((BPW, UD), jnp.float32),
        pltpu.VMEM((BPW, PD), jnp.float32),
        pltpu.SemaphoreType.DMA,
        pltpu.SemaphoreType.DMA,
    ],
)
def _user_model_sc(uid_hbm, gid_hbm, oid_hbm, utab_hbm, ptab_hbm, out_hbm,
                   uidx_v, pidx_v, oidx_v, urows_v, prows_v, sem_u, sem_p):
    wid = lax.axis_index("s") * NC + lax.axis_index("c")
    base = wid * BPW

    pltpu.sync_copy(uid_hbm.at[pl.ds(base, BPW)], uidx_v)
    cp_u = pltpu.async_copy(utab_hbm.at[uidx_v], urows_v, sem_u)

    pltpu.sync_copy(gid_hbm.at[pl.ds(base, BPW)], pidx_v)
    pltpu.sync_copy(oid_hbm.at[pl.ds(base, BPW)], oidx_v)

    def pair_body(i, _):
        g = pidx_v[pl.ds(i * L, L)]
        o = oidx_v[pl.ds(i * L, L)]
        pidx_v[pl.ds(i * L, L)] = g * N_OCC + o
        return ()

    lax.fori_loop(0, BPW // L, pair_body, (), unroll=4)

    cp_p = pltpu.async_copy(ptab_hbm.at[pidx_v], prows_v, sem_p)

    cp_u.wait()
    pltpu.sync_copy(urows_v, out_hbm.at[pl.ds(base, BPW), pl.ds(0, UD)])
    cp_p.wait()
    pltpu.sync_copy(prows_v, out_hbm.at[pl.ds(base, BPW), pl.ds(UD, PD)])


def kernel(user_id, gender, occupation, user_table, gender_table,
           occupation_table):
    n_g = gender_table.shape[0]
    n_o = occupation_table.shape[0]
    pairs = jnp.concatenate([
        jnp.broadcast_to(gender_table[:, None, :], (n_g, n_o, 8)),
        jnp.broadcast_to(occupation_table[None, :, :], (n_g, n_o, 8)),
    ], axis=-1).reshape(n_g * n_o, PD)
    return _user_model_sc(user_id, gender, occupation, user_table, pairs)


# speedup vs baseline: 2.1507x; 1.0625x over previous
import functools
import jax
import jax.numpy as jnp
from jax import lax
from jax.experimental import pallas as pl
from jax.experimental.pallas import tpu as pltpu
from jax.experimental.pallas import tpu_sc as plsc

NC, NS, L = 2, 16, 16
NW = NC * NS
B = 16384
BPW = B // NW
UD = 32
SD = 8
PD = 16

_mesh = plsc.VectorSubcoreMesh(core_axis_name="c", subcore_axis_name="s")


@functools.partial(
    pl.kernel,
    out_type=jax.ShapeDtypeStruct((B, UD + PD), jnp.float32),
    mesh=_mesh,
    compiler_params=pltpu.CompilerParams(use_tc_tiling_on_sc=False,
                                         needs_layout_passes=False),
    scratch_types=[
        pltpu.VMEM((BPW,), jnp.int32),
        pltpu.VMEM((BPW,), jnp.int32),
        pltpu.VMEM((BPW,), jnp.int32),
        pltpu.VMEM((BPW, UD), jnp.float32),
        pltpu.VMEM((BPW, PD), jnp.float32),
        pltpu.VMEM((32, SD), jnp.float32),
        pltpu.SemaphoreType.DMA,
    ],
)
def _user_model_sc(uid_hbm, gid_hbm, oid_hbm, utab_hbm, gtab_hbm, otab_hbm,
                   out_hbm, uidx_v, gidx_v, oidx_v, urows_v, go_v, small_v,
                   sem_u):
    wid = lax.axis_index("s") * NC + lax.axis_index("c")
    base = wid * BPW

    pltpu.sync_copy(uid_hbm.at[pl.ds(base, BPW)], uidx_v)
    cp_u = pltpu.async_copy(utab_hbm.at[uidx_v], urows_v, sem_u)

    pltpu.sync_copy(gid_hbm.at[pl.ds(base, BPW)], gidx_v)
    pltpu.sync_copy(oid_hbm.at[pl.ds(base, BPW)], oidx_v)

    pltpu.sync_copy(gtab_hbm, small_v.at[pl.ds(0, 3)])
    pltpu.sync_copy(otab_hbm, small_v.at[pl.ds(8, 22)])

    lanes = lax.iota(jnp.int32, L)

    def go_body(i, _):
        g = gidx_v[pl.ds(i * L, L)]
        o = oidx_v[pl.ds(i * L, L)] + 8
        rows = i * L + lanes
        for c in range(SD):
            cvec = jnp.full((L,), c, dtype=jnp.int32)
            vg = plsc.load_gather(small_v, [g, cvec])
            plsc.store_scatter(go_v, [rows, cvec], vg)
            vo = plsc.load_gather(small_v, [o, cvec])
            plsc.store_scatter(go_v, [rows, cvec + SD], vo)
        return ()

    lax.fori_loop(0, BPW // L, go_body, (), unroll=2)

    cp_u.wait()
    pltpu.sync_copy(urows_v, out_hbm.at[pl.ds(base, BPW), pl.ds(0, UD)])
    pltpu.sync_copy(go_v, out_hbm.at[pl.ds(base, BPW), pl.ds(UD, PD)])


def kernel(user_id, gender, occupation, user_table, gender_table,
           occupation_table):
    return _user_model_sc(user_id, gender, occupation, user_table,
                          gender_table, occupation_table)


# speedup vs baseline: 2.6811x; 1.2466x over previous
import functools
import jax
import jax.numpy as jnp
from jax import lax
from jax.experimental import pallas as pl
from jax.experimental.pallas import tpu as pltpu
from jax.experimental.pallas import tpu_sc as plsc

NC, NS, L = 2, 16, 16
NW = NC * NS
B = 16384
V = 100001
UD = 32
SD = 8
OD = UD + 2 * SD
CHUNK = 2048
N_CH = B // CHUNK
HALF = B // 2
N_SCH = HALF // CHUNK
STAGE = V + 7

_mesh = plsc.VectorSubcoreMesh(core_axis_name="c", subcore_axis_name="s")


@functools.partial(
    pl.kernel,
    out_type=jax.ShapeDtypeStruct((OD, B), jnp.float32),
    mesh=_mesh,
    compiler_params=pltpu.CompilerParams(use_tc_tiling_on_sc=False,
                                         needs_layout_passes=False),
    scratch_types=[
        pltpu.VMEM((STAGE,), jnp.float32),
        pltpu.VMEM((CHUNK,), jnp.int32),
        pltpu.VMEM((B,), jnp.float32),
        pltpu.VMEM((3, SD), jnp.float32),
        pltpu.VMEM((22, SD), jnp.float32),
        pltpu.VMEM((CHUNK,), jnp.int32),
        pltpu.VMEM((CHUNK,), jnp.float32),
        pltpu.SemaphoreType.DMA,
    ],
)
def _user_model_sc(uid_hbm, gid_hbm, oid_hbm, utabf_hbm, gtab_hbm, otab_hbm,
                   out_hbm, row_v, uidx_v, orow_v, gt_v, ot_v, sidx_v, srow_v,
                   sem_row):
    wid = lax.axis_index("s") * NC + lax.axis_index("c")

    row_begin = wid * V
    start = pl.multiple_of((row_begin // 8) * 8, 8)
    misal = row_begin - start
    cp_row = pltpu.async_copy(utabf_hbm.at[pl.ds(start, STAGE)], row_v,
                              sem_row)

    d = wid % 16
    col0 = (wid // 16) * HALF
    pltpu.sync_copy(gtab_hbm, gt_v)
    pltpu.sync_copy(otab_hbm, ot_v)

    def small_chunks(src_hbm, tab_v, dim):
        dvec = jnp.broadcast_to(dim, (L,))
        for k in range(N_SCH):
            pltpu.sync_copy(src_hbm.at[pl.ds(col0 + k * CHUNK, CHUNK)],
                            sidx_v)

            def body(j, _):
                ids = sidx_v[pl.ds(j * L, L)]
                srow_v[pl.ds(j * L, L)] = plsc.load_gather(tab_v, [ids, dvec])
                return ()

            lax.fori_loop(0, CHUNK // L, body, (), unroll=8)
            pltpu.sync_copy(
                srow_v, out_hbm.at[UD + d, pl.ds(col0 + k * CHUNK, CHUNK)])

    @pl.when(d < SD)
    def _gender():
        small_chunks(gid_hbm, gt_v, d)

    @pl.when(d >= SD)
    def _occ():
        small_chunks(oid_hbm, ot_v, d - SD)

    cp_row.wait()
    for k in range(N_CH):
        pltpu.sync_copy(uid_hbm.at[pl.ds(k * CHUNK, CHUNK)], uidx_v)

        def mbody(j, _):
            ids = uidx_v[pl.ds(j * L, L)] + misal
            orow_v[pl.ds(k * CHUNK + j * L, L)] = plsc.load_gather(
                row_v, [ids])
            return ()

        lax.fori_loop(0, CHUNK // L, mbody, (), unroll=8)
    pltpu.sync_copy(orow_v, out_hbm.at[wid])


def kernel(user_id, gender, occupation, user_table, gender_table,
           occupation_table):
    utab_flat = user_table.T.reshape(UD * V)
    out_t = _user_model_sc(user_id, gender, occupation, utab_flat,
                           gender_table, occupation_table)
    return out_t.T


# speedup vs baseline: 2.9651x; 1.1059x over previous
import functools
import jax
import jax.numpy as jnp
from jax import lax
from jax.experimental import pallas as pl
from jax.experimental.pallas import tpu as pltpu
from jax.experimental.pallas import tpu_sc as plsc

NC, NS, L = 2, 16, 16
NW = NC * NS
B = 16384
V = 100001
UD = 32
SD = 8
OD = UD + 2 * SD
CHUNK = 4096
N_CH = B // CHUNK
HALF = B // 2
N_SCH = HALF // CHUNK
STAGE = V + 7

_mesh = plsc.VectorSubcoreMesh(core_axis_name="c", subcore_axis_name="s")


@functools.partial(
    pl.kernel,
    out_type=jax.ShapeDtypeStruct((OD, B), jnp.float32),
    mesh=_mesh,
    compiler_params=pltpu.CompilerParams(use_tc_tiling_on_sc=False,
                                         needs_layout_passes=False),
    scratch_types=[
        pltpu.VMEM((STAGE,), jnp.float32),
        pltpu.VMEM((2 * CHUNK,), jnp.int32),
        pltpu.VMEM((B,), jnp.float32),
        pltpu.VMEM((3, SD), jnp.float32),
        pltpu.VMEM((22, SD), jnp.float32),
        pltpu.SemaphoreType.DMA,
        pltpu.SemaphoreType.DMA,
        pltpu.SemaphoreType.DMA,
    ],
)
def _user_model_sc(uid_hbm, gid_hbm, oid_hbm, utabf_hbm, gtab_hbm, otab_hbm,
                   out_hbm, row_v, idx_v, orow_v, gt_v, ot_v,
                   sem_row, sem_a, sem_b):
    wid = lax.axis_index("s") * NC + lax.axis_index("c")
    sems = (sem_a, sem_b)

    row_begin = wid * V
    start = pl.multiple_of((row_begin // 8) * 8, 8)
    misal = row_begin - start
    cp_row = pltpu.async_copy(utabf_hbm.at[pl.ds(start, STAGE)], row_v,
                              sem_row)

    d = wid % 16
    col0 = (wid // 16) * HALF
    pltpu.sync_copy(gtab_hbm, gt_v)
    pltpu.sync_copy(otab_hbm, ot_v)

    def gather_chunks(src_hbm, src0, n_ch, out_off, value_fn):
        cps = [None, None]
        cps[0] = pltpu.async_copy(
            src_hbm.at[pl.ds(src0, CHUNK)], idx_v.at[pl.ds(0, CHUNK)],
            sems[0])
        for k in range(n_ch):
            p = k % 2
            cps[p].wait()
            if k + 1 < n_ch:
                q = (k + 1) % 2
                cps[q] = pltpu.async_copy(
                    src_hbm.at[pl.ds(src0 + (k + 1) * CHUNK, CHUNK)],
                    idx_v.at[pl.ds(q * CHUNK, CHUNK)], sems[q])

            def body(j, _):
                ids = idx_v[pl.ds(p * CHUNK + j * L, L)]
                orow_v[pl.ds(out_off + k * CHUNK + j * L, L)] = value_fn(ids)
                return ()

            lax.fori_loop(0, CHUNK // L, body, (), unroll=8)

    @pl.when(d < SD)
    def _gender():
        dvec = jnp.broadcast_to(d, (L,))
        gather_chunks(gid_hbm, col0, N_SCH, 0,
                      lambda ids: plsc.load_gather(gt_v, [ids, dvec]))

    @pl.when(d >= SD)
    def _occ():
        dvec = jnp.broadcast_to(d - SD, (L,))
        gather_chunks(oid_hbm, col0, N_SCH, 0,
                      lambda ids: plsc.load_gather(ot_v, [ids, dvec]))

    pltpu.sync_copy(orow_v.at[pl.ds(0, HALF)],
                    out_hbm.at[UD + d, pl.ds(col0, HALF)])

    cp_row.wait()
    gather_chunks(uid_hbm, 0, N_CH, 0,
                  lambda ids: plsc.load_gather(row_v, [ids + misal]))
    pltpu.sync_copy(orow_v, out_hbm.at[wid])


def kernel(user_id, gender, occupation, user_table, gender_table,
           occupation_table):
    utab_flat = user_table.T.reshape(UD * V)
    out_t = _user_model_sc(user_id, gender, occupation, utab_flat,
                           gender_table, occupation_table)
    return out_t.T


# speedup vs baseline: 3.5818x; 1.2080x over previous
import functools
import jax
import jax.numpy as jnp
from jax import lax
from jax.experimental import pallas as pl
from jax.experimental.pallas import tpu as pltpu
from jax.experimental.pallas import tpu_sc as plsc

NC, NS, L = 2, 16, 16
NW = NC * NS
B = 16384
V = 100001
UD = 32
SD = 8
OD = UD + 2 * SD
CHUNK = 4096
N_CH = B // CHUNK
HALF = B // 2
STAGE = V + 7

_mesh = plsc.VectorSubcoreMesh(core_axis_name="c", subcore_axis_name="s")


@functools.partial(
    pl.kernel,
    out_type=jax.ShapeDtypeStruct((OD, B), jnp.float32),
    mesh=_mesh,
    compiler_params=pltpu.CompilerParams(use_tc_tiling_on_sc=False,
                                         needs_layout_passes=False),
    scratch_types=[
        pltpu.VMEM((STAGE,), jnp.float32),
        pltpu.VMEM((2 * CHUNK,), jnp.int32),
        pltpu.VMEM((HALF,), jnp.int32),
        pltpu.VMEM((2 * CHUNK,), jnp.float32),
        pltpu.VMEM((3, SD), jnp.float32),
        pltpu.VMEM((22, SD), jnp.float32),
        pltpu.SemaphoreType.DMA,
        pltpu.SemaphoreType.DMA,
        pltpu.SemaphoreType.DMA,
        pltpu.SemaphoreType.DMA,
        pltpu.SemaphoreType.DMA,
    ],
)
def _user_model_sc(uid_hbm, gid_hbm, oid_hbm, utabf_hbm, gtab_hbm, otab_hbm,
                   out_hbm, row_v, idx_v, sidx_v, obuf_v, gt_v, ot_v,
                   sem_row, sem_u0, sem_u1, sem_s, sem_o):
    wid = lax.axis_index("s") * NC + lax.axis_index("c")
    sems_u = (sem_u0, sem_u1)

    row_begin = wid * V
    start = pl.multiple_of((row_begin // 8) * 8, 8)
    misal = row_begin - start
    cp_row = pltpu.async_copy(utabf_hbm.at[pl.ds(start, STAGE)], row_v,
                              sem_row)

    cp_u = [None] * N_CH
    for k in range(2):
        cp_u[k] = pltpu.async_copy(uid_hbm.at[pl.ds(k * CHUNK, CHUNK)],
                                   idx_v.at[pl.ds(k * CHUNK, CHUNK)],
                                   sems_u[k])

    d = wid % 16
    col0 = (wid // 16) * HALF
    pltpu.sync_copy(gtab_hbm, gt_v)
    pltpu.sync_copy(otab_hbm, ot_v)

    def small(src_hbm, tab_v, dim):
        pltpu.async_copy(src_hbm.at[pl.ds(col0, HALF)], sidx_v, sem_s).wait()
        dvec = jnp.broadcast_to(dim, (L,))

        def body(j, _):
            ids = sidx_v[pl.ds(j * L, L)]
            obuf_v[pl.ds(j * L, L)] = plsc.load_gather(tab_v, [ids, dvec])
            return ()

        lax.fori_loop(0, HALF // L, body, (), unroll=16)

    @pl.when(d < SD)
    def _gender():
        small(gid_hbm, gt_v, d)

    @pl.when(d >= SD)
    def _occ():
        small(oid_hbm, ot_v, d - SD)

    cp_so = pltpu.async_copy(obuf_v, out_hbm.at[UD + d, pl.ds(col0, HALF)],
                             sem_o)

    cp_row.wait()
    cp_so.wait()
    cp_o = [None] * N_CH
    for k in range(N_CH):
        p = k % 2
        cp_u[k].wait()
        if k >= 2:
            cp_o[k - 2].wait()

        def mbody(j, _):
            ids = idx_v[pl.ds(p * CHUNK + j * L, L)] + misal
            obuf_v[pl.ds(p * CHUNK + j * L, L)] = plsc.load_gather(
                row_v, [ids])
            return ()

        lax.fori_loop(0, CHUNK // L, mbody, (), unroll=16)
        cp_o[k] = pltpu.async_copy(obuf_v.at[pl.ds(p * CHUNK, CHUNK)],
                                   out_hbm.at[wid, pl.ds(k * CHUNK, CHUNK)],
                                   sem_o)
        if k + 2 < N_CH:
            cp_u[k + 2] = pltpu.async_copy(
                uid_hbm.at[pl.ds((k + 2) * CHUNK, CHUNK)],
                idx_v.at[pl.ds(p * CHUNK, CHUNK)], sems_u[p])
    cp_o[N_CH - 2].wait()
    cp_o[N_CH - 1].wait()


def kernel(user_id, gender, occupation, user_table, gender_table,
           occupation_table):
    utab_flat = user_table.T.reshape(UD * V)
    out_t = _user_model_sc(user_id, gender, occupation, utab_flat,
                           gender_table, occupation_table)
    return out_t.T
